# baseline (speedup 1.0000x reference)
import functools

import jax
import jax.numpy as jnp
from jax import lax
from jax.experimental import pallas as pl
from jax.experimental.pallas import tpu as pltpu
from jax.experimental.pallas import tpu_sc as plsc

B = 1024
D = 512
KNN = 100
NH = 10
NV = NH + 1
NIMG = 100000
NC = 2
NS = 16
NW = NC * NS
BPW = B // NW
CB = 8
NCHUNK = BPW // CB
ROWS = CB * NH
DK = D // 16

_mesh = plsc.VectorSubcoreMesh(core_axis_name="c", subcore_axis_name="s")


@functools.partial(
    pl.kernel,
    out_type=(
        jax.ShapeDtypeStruct((B, D), jnp.float32),
        jax.ShapeDtypeStruct((B * 16,), jnp.float32),
    ),
    mesh=_mesh,
    compiler_params=pltpu.CompilerParams(needs_layout_passes=False),
    scratch_types=[
        pltpu.VMEM((BPW,), jnp.int32),
        pltpu.VMEM((BPW,), jnp.int32),
        pltpu.VMEM((BPW,), jnp.int32),
        pltpu.VMEM((BPW * NH,), jnp.int32),
        pltpu.VMEM((BPW * NH,), jnp.int32),
        pltpu.VMEM((BPW, D), jnp.float32),
        pltpu.VMEM((BPW, D), jnp.float32),
        pltpu.VMEM((ROWS, D), jnp.float32),
        pltpu.VMEM((ROWS, D), jnp.float32),
        pltpu.VMEM((ROWS * 16,), jnp.float32),
        pltpu.VMEM((BPW * 16,), jnp.float32),
        pltpu.SemaphoreType.DMA,
        pltpu.SemaphoreType.DMA,
        pltpu.SemaphoreType.DMA,
        pltpu.SemaphoreType.DMA,
        [pltpu.SemaphoreType.DMA] * NCHUNK,
    ],
)
def _sc_gather_dots(elem_hbm, ncols_hbm, text_hbm, emb_hbm, pos_hbm, dots_hbm,
                    idx_v, pnidx_v, pidx_v, nnidx_v, eidx_v,
                    text_v, pos_v, gath0, gath1, part_v, dots_v,
                    sem0, sem1, tsem, psem, vsems):
    wid = lax.axis_index("s") * NC + lax.axis_index("c")
    base = wid * BPW
    lanes = jnp.arange(16, dtype=jnp.int32)

    pltpu.sync_copy(elem_hbm.at[pl.ds(base, BPW)], idx_v)
    tcopy = pltpu.async_copy(text_hbm.at[pl.ds(base, BPW)], text_v, tsem)

    for h in range(BPW // 16):
        ev = plsc.load_gather(idx_v, [lanes + h * 16])
        pnidx_v[pl.ds(h * 16, 16)] = ev
    pvcopy = pltpu.async_copy(ncols_hbm.at[pnidx_v], pidx_v, psem)

    for g in range((BPW * NH) // 16):
        p = lanes + (g * 16)
        bv = p // NH
        jn = p - bv * NH
        ev = plsc.load_gather(idx_v, [bv])
        nnidx_v[pl.ds(g * 16, 16)] = (jn + 1) * NIMG + ev

    ncopies = [
        pltpu.async_copy(ncols_hbm.at[nnidx_v.at[pl.ds(q * ROWS, ROWS)]],
                         eidx_v.at[pl.ds(q * ROWS, ROWS)], vsems[q])
        for q in range(NCHUNK)
    ]
    pvcopy.wait()
    pcopy = pltpu.async_copy(emb_hbm.at[pidx_v], pos_v, psem)

    bufs = (gath0, gath1)
    sems = (sem0, sem1)

    def issue(c):
        return pltpu.async_copy(
            emb_hbm.at[eidx_v.at[pl.ds(c * ROWS, ROWS)]], bufs[c % 2], sems[c % 2])

    ncopies[0].wait()
    copies = [issue(0), None]
    tcopy.wait()
    for c in range(NCHUNK):
        if c + 1 < NCHUNK:
            ncopies[c + 1].wait()
            copies[(c + 1) % 2] = issue(c + 1)
        copies[c % 2].wait()
        gath = bufs[c % 2]

        def bbody(lb, carry, gath=gath, c=c):
            def kbody(k, accs, lb=lb):
                t = text_v[c * CB + lb, pl.ds(k * 16, 16)]
                return tuple(accs[j] + gath[lb * NH + j, pl.ds(k * 16, 16)] * t
                             for j in range(NH))
            accs = lax.fori_loop(
                0, DK, kbody,
                tuple(jnp.zeros((16,), jnp.float32) for _ in range(NH)),
                unroll=4)
            for j in range(NH):
                part_v[pl.ds((lb * NH + j) * 16, 16)] = accs[j]
            return carry
        lax.fori_loop(0, CB, bbody, 0)

        for g in range(ROWS // 16):
            p = lanes + (g * 16)
            tot = jnp.zeros((16,), jnp.float32)
            for col in range(16):
                tot = tot + plsc.load_gather(part_v, [p * 16 + col])
            lb = p // NH
            jv = p - lb * NH + 1
            plsc.store_scatter(dots_v, [jv * BPW + (lb + c * CB)], tot)

    pcopy.wait()
    pltpu.sync_copy(pos_v, pos_hbm.at[pl.ds(base, BPW)])
    for jv in range(1, NH + 1):
        pltpu.sync_copy(dots_v.at[pl.ds(jv * BPW, BPW)],
                        dots_hbm.at[pl.ds(jv * B + base, BPW)])


def _tc_body(text_ref, pos_ref, dots_ref, out_ref):
    t = text_ref[...]
    inv = lax.rsqrt(jnp.sum(t * t, axis=1, keepdims=True))
    tn = t * inv
    raw_t = lax.dot_general(pos_ref[...], tn,
                            dimension_numbers=(((1,), (1,)), ((), ())),
                            preferred_element_type=jnp.float32)
    r = lax.broadcasted_iota(jnp.int32, (B, B), 0)
    c = lax.broadcasted_iota(jnp.int32, (B, B), 1)
    eye = r == c
    diag = jnp.sum(jnp.where(eye, raw_t, 0.0), axis=0, keepdims=True)
    inv_row = jnp.reshape(inv, (1, B))
    neg_t = jnp.reshape(dots_ref[...], (16, B)) * inv_row
    row = lax.broadcasted_iota(jnp.int32, (16, B), 0)
    neg_t = jnp.where(row == 0, diag, neg_t)
    out_ref[pl.ds(0, NV), :] = neg_t[:NV, :]
    out_ref[pl.ds(NV, B), :] = jnp.where(eye, -jnp.inf, raw_t)


def kernel(elem_idxs, text_feats, emb, nns):
    elem_idxs = jnp.squeeze(elem_idxs).astype(jnp.int32)
    ncols = jnp.reshape(jnp.transpose(nns)[:NV], (-1,))
    pos, dots = _sc_gather_dots(elem_idxs, ncols, text_feats, emb)
    out_t = pl.pallas_call(
        _tc_body,
        out_shape=jax.ShapeDtypeStruct((NV + B, B), jnp.float32),
    )(text_feats, pos, dots)
    return jnp.transpose(out_t)


# speedup vs baseline: 1.0096x; 1.0096x over previous
import functools

import jax
import jax.numpy as jnp
from jax import lax
from jax.experimental import pallas as pl
from jax.experimental.pallas import tpu as pltpu
from jax.experimental.pallas import tpu_sc as plsc

B = 1024
D = 512
KNN = 100
NH = 10
NV = NH + 1
NIMG = 100000
NC = 2
NS = 16
NW = NC * NS
BPW = B // NW
CB = 8
NCHUNK = BPW // CB
ROWS = CB * NH
DK = D // 16

_mesh = plsc.VectorSubcoreMesh(core_axis_name="c", subcore_axis_name="s")


@functools.partial(
    pl.kernel,
    out_type=(
        jax.ShapeDtypeStruct((B, D), jnp.float32),
        jax.ShapeDtypeStruct((B * 16,), jnp.float32),
    ),
    mesh=_mesh,
    compiler_params=pltpu.CompilerParams(needs_layout_passes=False),
    scratch_types=[
        pltpu.VMEM((BPW,), jnp.int32),
        pltpu.VMEM((BPW,), jnp.int32),
        pltpu.VMEM((BPW,), jnp.int32),
        pltpu.VMEM((BPW * NH,), jnp.int32),
        pltpu.VMEM((BPW * NH,), jnp.int32),
        pltpu.VMEM((BPW, D), jnp.float32),
        pltpu.VMEM((BPW, D), jnp.float32),
        pltpu.VMEM((ROWS, D), jnp.float32),
        pltpu.VMEM((ROWS, D), jnp.float32),
        pltpu.VMEM((ROWS * 16,), jnp.float32),
        pltpu.VMEM((BPW * 16,), jnp.float32),
        pltpu.SemaphoreType.DMA,
        pltpu.SemaphoreType.DMA,
        pltpu.SemaphoreType.DMA,
        pltpu.SemaphoreType.DMA,
        [pltpu.SemaphoreType.DMA] * NCHUNK,
    ],
)
def _sc_gather_dots(elem_hbm, ncols_hbm, text_hbm, emb_hbm, pos_hbm, dots_hbm,
                    idx_v, pnidx_v, pidx_v, nnidx_v, eidx_v,
                    text_v, pos_v, gath0, gath1, part_v, dots_v,
                    sem0, sem1, tsem, psem, vsems):
    wid = lax.axis_index("s") * NC + lax.axis_index("c")
    base = wid * BPW
    lanes = jnp.arange(16, dtype=jnp.int32)

    pltpu.sync_copy(elem_hbm.at[pl.ds(base, BPW)], idx_v)
    tcopy = pltpu.async_copy(text_hbm.at[pl.ds(base, BPW)], text_v, tsem)

    for h in range(BPW // 16):
        ev = plsc.load_gather(idx_v, [lanes + h * 16])
        pnidx_v[pl.ds(h * 16, 16)] = ev
    pvcopy = pltpu.async_copy(ncols_hbm.at[pnidx_v], pidx_v, psem)

    for g in range((BPW * NH) // 16):
        p = lanes + (g * 16)
        bv = p // NH
        jn = p - bv * NH
        ev = plsc.load_gather(idx_v, [bv])
        nnidx_v[pl.ds(g * 16, 16)] = (jn + 1) * NIMG + ev

    ncopies = [
        pltpu.async_copy(ncols_hbm.at[nnidx_v.at[pl.ds(q * ROWS, ROWS)]],
                         eidx_v.at[pl.ds(q * ROWS, ROWS)], vsems[q])
        for q in range(NCHUNK)
    ]
    pvcopy.wait()
    pcopy = pltpu.async_copy(emb_hbm.at[pidx_v], pos_v, psem)

    bufs = (gath0, gath1)
    sems = (sem0, sem1)

    def issue(c):
        return pltpu.async_copy(
            emb_hbm.at[eidx_v.at[pl.ds(c * ROWS, ROWS)]], bufs[c % 2], sems[c % 2])

    ncopies[0].wait()
    copies = [issue(0), None]
    tcopy.wait()
    for c in range(NCHUNK):
        if c + 1 < NCHUNK:
            ncopies[c + 1].wait()
            copies[(c + 1) % 2] = issue(c + 1)
        copies[c % 2].wait()
        gath = bufs[c % 2]

        def bbody(lb, carry, gath=gath, c=c):
            def kbody(k, accs, lb=lb):
                t = text_v[c * CB + lb, pl.ds(k * 16, 16)]
                return tuple(accs[j] + gath[lb * NH + j, pl.ds(k * 16, 16)] * t
                             for j in range(NH))
            accs = lax.fori_loop(
                0, DK, kbody,
                tuple(jnp.zeros((16,), jnp.float32) for _ in range(NH)),
                unroll=4)
            for j in range(NH):
                part_v[pl.ds((lb * NH + j) * 16, 16)] = accs[j]
            return carry
        lax.fori_loop(0, CB, bbody, 0)

        for g in range(ROWS // 16):
            p = lanes + (g * 16)
            tot = jnp.zeros((16,), jnp.float32)
            for col in range(16):
                tot = tot + plsc.load_gather(part_v, [p * 16 + col])
            lb = p // NH
            jv = p - lb * NH + 1
            plsc.store_scatter(dots_v, [jv * BPW + (lb + c * CB)], tot)

    pcopy.wait()
    dcopies = [
        pltpu.async_copy(dots_v.at[pl.ds(jv * BPW, BPW)],
                         dots_hbm.at[pl.ds(jv * B + base, BPW)], tsem)
        for jv in range(1, NH + 1)
    ]
    pwcopy = pltpu.async_copy(pos_v, pos_hbm.at[pl.ds(base, BPW)], psem)
    for cp in dcopies:
        cp.wait()
    pwcopy.wait()


def _tc_body(text_ref, pos_ref, dots_ref, out_ref):
    t = text_ref[...]
    inv = lax.rsqrt(jnp.sum(t * t, axis=1, keepdims=True))
    tn = t * inv
    raw_t = lax.dot_general(pos_ref[...], tn,
                            dimension_numbers=(((1,), (1,)), ((), ())),
                            preferred_element_type=jnp.float32)
    r = lax.broadcasted_iota(jnp.int32, (B, B), 0)
    c = lax.broadcasted_iota(jnp.int32, (B, B), 1)
    eye = r == c
    diag = jnp.sum(jnp.where(eye, raw_t, 0.0), axis=0, keepdims=True)
    inv_row = jnp.reshape(inv, (1, B))
    neg_t = jnp.reshape(dots_ref[...], (16, B)) * inv_row
    row = lax.broadcasted_iota(jnp.int32, (16, B), 0)
    neg_t = jnp.where(row == 0, diag, neg_t)
    out_ref[pl.ds(0, NV), :] = neg_t[:NV, :]
    out_ref[pl.ds(NV, B), :] = jnp.where(eye, -jnp.inf, raw_t)


def kernel(elem_idxs, text_feats, emb, nns):
    elem_idxs = jnp.squeeze(elem_idxs).astype(jnp.int32)
    ncols = jnp.reshape(jnp.transpose(nns)[:NV], (-1,))
    pos, dots = _sc_gather_dots(elem_idxs, ncols, text_feats, emb)
    out_t = pl.pallas_call(
        _tc_body,
        out_shape=jax.ShapeDtypeStruct((NV + B, B), jnp.float32),
    )(text_feats, pos, dots)
    return jnp.transpose(out_t)
